# baseline (speedup 1.0000x reference)
import functools

import jax
import jax.numpy as jnp
from jax import lax
from jax.experimental import pallas as pl
from jax.experimental.pallas import tpu as pltpu
from jax.experimental.pallas import tpu_sc as plsc

B, CTX, D, V = 1024, 20, 64, 100000

NC, NS, L = 2, 16, 16
NW = NC * NS
BPW = B // NW
RPW = BPW * CTX
CHUNK = 128
NCHUNK = RPW // CHUNK

V_TILE = 2048
NV = (V + V_TILE - 1) // V_TILE
VP = NV * V_TILE


DP = 128


def _sc_embed_sum(x3, embp):
    mesh = plsc.VectorSubcoreMesh(core_axis_name="c", subcore_axis_name="s")

    @functools.partial(
        pl.kernel,
        mesh=mesh,
        out_type=jax.ShapeDtypeStruct((B, D), jnp.float32),
        scratch_types=[
            pltpu.VMEM((NCHUNK, CHUNK), jnp.int32),
            pltpu.VMEM((RPW, DP), jnp.float32),
            pltpu.VMEM((BPW, D), jnp.float32),
            pltpu.SemaphoreType.DMA,
        ],
    )
    def k(x_hbm, emb_hbm, out_hbm, idx_v, rows_v, acc_v, sem):
        wid = lax.axis_index("s") * NC + lax.axis_index("c")
        pltpu.sync_copy(x_hbm.at[wid], idx_v)
        copies = [
            pltpu.async_copy(
                emb_hbm.at[idx_v.at[c]],
                rows_v.at[pl.ds(c * CHUNK, CHUNK)],
                sem,
            )
            for c in range(NCHUNK)
        ]
        for cp in copies:
            cp.wait()

        def body(bi, carry):
            base = bi * CTX
            for g in range(D // L):
                acc = rows_v[base, pl.ds(g * L, L)]
                for j in range(1, CTX):
                    acc = acc + rows_v[base + j, pl.ds(g * L, L)]
                acc_v[bi, pl.ds(g * L, L)] = acc
            return carry

        lax.fori_loop(0, BPW, body, 0)
        pltpu.sync_copy(acc_v, out_hbm.at[pl.ds(wid * BPW, BPW)])

    return k(x3, embp)


def _tc_stats(s, x32, emb0, Wp, bp):
    def kern(s_ref, x_ref, e0_ref, w_ref, b_ref, m_ref, lse_ref, mrun, lrun):
        v = pl.program_id(0)

        @pl.when(v == 0)
        def _():
            n0 = jnp.sum(
                jnp.where(x_ref[...] == 0, 1.0, 0.0), axis=1, keepdims=True
            )
            m_ref[...] = (s_ref[...] - n0 * e0_ref[...]) * (1.0 / CTX)
            mrun[...] = jnp.full((B, 1), -1e30, jnp.float32)
            lrun[...] = jnp.zeros((B, 1), jnp.float32)

        logits = (
            lax.dot_general(
                m_ref[...],
                w_ref[...],
                (((1,), (1,)), ((), ())),
                preferred_element_type=jnp.float32,
            )
            + b_ref[...]
        )
        tmax = jnp.max(logits, axis=1, keepdims=True)
        mold = mrun[...]
        mnew = jnp.maximum(mold, tmax)
        lrun[...] = lrun[...] * jnp.exp(mold - mnew) + jnp.sum(
            jnp.exp(logits - mnew), axis=1, keepdims=True
        )
        mrun[...] = mnew

        @pl.when(v == NV - 1)
        def _():
            lse_ref[...] = mrun[...] + jnp.log(lrun[...])

    return pl.pallas_call(
        kern,
        grid=(NV,),
        in_specs=[
            pl.BlockSpec((B, D), lambda v: (0, 0)),
            pl.BlockSpec((B, CTX), lambda v: (0, 0)),
            pl.BlockSpec((1, D), lambda v: (0, 0)),
            pl.BlockSpec((V_TILE, D), lambda v: (v, 0)),
            pl.BlockSpec((1, V_TILE), lambda v: (0, v)),
        ],
        out_specs=[
            pl.BlockSpec((B, D), lambda v: (0, 0)),
            pl.BlockSpec((B, 1), lambda v: (0, 0)),
        ],
        out_shape=[
            jax.ShapeDtypeStruct((B, D), jnp.float32),
            jax.ShapeDtypeStruct((B, 1), jnp.float32),
        ],
        scratch_shapes=[
            pltpu.VMEM((B, 1), jnp.float32),
            pltpu.VMEM((B, 1), jnp.float32),
        ],
    )(s, x32, emb0, Wp, bp)


def _tc_out(m, Wp, bp, lse):
    def kern(m_ref, w_ref, b_ref, l_ref, o_ref):
        logits = (
            lax.dot_general(
                m_ref[...],
                w_ref[...],
                (((1,), (1,)), ((), ())),
                preferred_element_type=jnp.float32,
            )
            + b_ref[...]
        )
        o_ref[...] = logits - l_ref[...]

    return pl.pallas_call(
        kern,
        grid=(NV,),
        in_specs=[
            pl.BlockSpec((B, D), lambda v: (0, 0)),
            pl.BlockSpec((V_TILE, D), lambda v: (v, 0)),
            pl.BlockSpec((1, V_TILE), lambda v: (0, v)),
            pl.BlockSpec((B, 1), lambda v: (0, 0)),
        ],
        out_specs=pl.BlockSpec((B, V_TILE), lambda v: (0, v)),
        out_shape=jax.ShapeDtypeStruct((B, V), jnp.float32),
    )(m, Wp, bp, lse)


def kernel(x, emb, W, b):
    x32 = x.astype(jnp.int32)
    embp = jnp.pad(emb, ((0, 0), (0, DP - D)))
    s = _sc_embed_sum(x32.reshape(NW, NCHUNK, CHUNK), embp)
    pad = VP - V
    Wp = jnp.concatenate([W, jnp.zeros((pad, D), W.dtype)], axis=0)
    bp = jnp.concatenate(
        [b, jnp.full((pad,), -1e30, b.dtype)], axis=0
    ).reshape(1, VP)
    m, lse = _tc_stats(s, x32, emb[0:1, :], Wp, bp)
    out = _tc_out(m, Wp, bp, lse)
    return out[:, None, :]


# speedup vs baseline: 1.1062x; 1.1062x over previous
import functools

import jax
import jax.numpy as jnp
from jax import lax
from jax.experimental import pallas as pl
from jax.experimental.pallas import tpu as pltpu
from jax.experimental.pallas import tpu_sc as plsc

B, CTX, D, V = 1024, 20, 64, 100000

NC, NS, L = 2, 16, 16
NW = NC * NS
BPW = B // NW
RPW = BPW * CTX
DP = 128

V_TILE = 2048
NV = (V + V_TILE - 1) // V_TILE
LAST_VALID = V - (NV - 1) * V_TILE

PB = 10000
NP = V // PB


def _tc_pad(emb):
    def kern(e_ref, o_ref):
        o_ref[...] = jnp.concatenate(
            [e_ref[...], jnp.zeros((PB, DP - D), jnp.float32)], axis=1
        )

    return pl.pallas_call(
        kern,
        grid=(NP,),
        in_specs=[pl.BlockSpec((PB, D), lambda i: (i, 0))],
        out_specs=pl.BlockSpec((PB, DP), lambda i: (i, 0)),
        out_shape=jax.ShapeDtypeStruct((V, DP), jnp.float32),
    )(emb)


def _sc_embed_sum(x32, embp):
    mesh = plsc.VectorSubcoreMesh(core_axis_name="c", subcore_axis_name="s")

    @functools.partial(
        pl.kernel,
        mesh=mesh,
        out_type=jax.ShapeDtypeStruct((B, D), jnp.float32),
        scratch_types=[
            pltpu.VMEM((BPW, CTX), jnp.int32),
            pltpu.VMEM((RPW, DP), jnp.float32),
            pltpu.VMEM((BPW, D), jnp.float32),
            pltpu.SemaphoreType.DMA,
        ],
    )
    def k(x_hbm, emb_hbm, out_hbm, idx_v, rows_v, acc_v, sem):
        wid = lax.axis_index("s") * NC + lax.axis_index("c")
        base = wid * BPW
        pltpu.sync_copy(x_hbm.at[pl.ds(base, BPW)], idx_v)
        for half in range(2):
            lo = half * (BPW // 2)
            copies = [
                pltpu.async_copy(
                    emb_hbm.at[idx_v.at[lo + i]],
                    rows_v.at[pl.ds((lo + i) * CTX, CTX)],
                    sem,
                )
                for i in range(BPW // 2)
            ]
            for cp in copies:
                cp.wait()

        def body(bi, carry):
            rbase = bi * CTX
            for g in range(D // L):
                acc = rows_v[rbase, pl.ds(g * L, L)]
                for j in range(1, CTX):
                    acc = acc + rows_v[rbase + j, pl.ds(g * L, L)]
                acc_v[bi, pl.ds(g * L, L)] = acc
            return carry

        lax.fori_loop(0, BPW, body, 0)
        pltpu.sync_copy(acc_v, out_hbm.at[pl.ds(base, BPW)])

    return k(x32, embp)


def _tc_stats(s, x32, emb0, W, b2):
    def kern(s_ref, x_ref, e0_ref, w_ref, b_ref, m_ref, lse_ref, lrun):
        v = pl.program_id(0)

        @pl.when(v == 0)
        def _():
            n0 = jnp.sum(
                jnp.where(x_ref[...] == 0, 1.0, 0.0), axis=1, keepdims=True
            )
            m_ref[...] = (s_ref[...] - n0 * e0_ref[...]) * (1.0 / CTX)
            lrun[...] = jnp.zeros((B, 1), jnp.float32)

        logits = (
            lax.dot_general(
                m_ref[...],
                w_ref[...],
                (((1,), (1,)), ((), ())),
                preferred_element_type=jnp.float32,
            )
            + b_ref[...]
        )
        ex = jnp.exp(logits)

        @pl.when(v < NV - 1)
        def _():
            lrun[...] = lrun[...] + jnp.sum(ex, axis=1, keepdims=True)

        @pl.when(v == NV - 1)
        def _():
            col = lax.broadcasted_iota(jnp.int32, (1, V_TILE), 1)
            exm = jnp.where(col < LAST_VALID, ex, 0.0)
            lse_ref[...] = jnp.log(
                lrun[...] + jnp.sum(exm, axis=1, keepdims=True)
            )

    return pl.pallas_call(
        kern,
        grid=(NV,),
        in_specs=[
            pl.BlockSpec((B, D), lambda v: (0, 0)),
            pl.BlockSpec((B, CTX), lambda v: (0, 0)),
            pl.BlockSpec((1, D), lambda v: (0, 0)),
            pl.BlockSpec((V_TILE, D), lambda v: (v, 0)),
            pl.BlockSpec((1, V_TILE), lambda v: (0, v)),
        ],
        out_specs=[
            pl.BlockSpec((B, D), lambda v: (0, 0)),
            pl.BlockSpec((B, 1), lambda v: (0, 0)),
        ],
        out_shape=[
            jax.ShapeDtypeStruct((B, D), jnp.float32),
            jax.ShapeDtypeStruct((B, 1), jnp.float32),
        ],
        scratch_shapes=[pltpu.VMEM((B, 1), jnp.float32)],
    )(s, x32, emb0, W, b2)


def _tc_out(m, W, b2, lse):
    def kern(m_ref, w_ref, b_ref, l_ref, o_ref):
        logits = (
            lax.dot_general(
                m_ref[...],
                w_ref[...],
                (((1,), (1,)), ((), ())),
                preferred_element_type=jnp.float32,
            )
            + b_ref[...]
        )
        o_ref[...] = logits - l_ref[...]

    return pl.pallas_call(
        kern,
        grid=(NV,),
        in_specs=[
            pl.BlockSpec((B, D), lambda v: (0, 0)),
            pl.BlockSpec((V_TILE, D), lambda v: (v, 0)),
            pl.BlockSpec((1, V_TILE), lambda v: (0, v)),
            pl.BlockSpec((B, 1), lambda v: (0, 0)),
        ],
        out_specs=pl.BlockSpec((B, V_TILE), lambda v: (0, v)),
        out_shape=jax.ShapeDtypeStruct((B, V), jnp.float32),
    )(m, W, b2, lse)


def kernel(x, emb, W, b):
    x32 = x.astype(jnp.int32)
    embp = _tc_pad(emb)
    s = _sc_embed_sum(x32, embp)
    b2 = b[None, :]
    m, lse = _tc_stats(s, x32, emb[0:1, :], W, b2)
    out = _tc_out(m, W, b2, lse)
    return out[:, None, :]


# speedup vs baseline: 2.1771x; 1.9681x over previous
import functools

import jax
import jax.numpy as jnp
from jax import lax
from jax.experimental import pallas as pl
from jax.experimental.pallas import tpu as pltpu
from jax.experimental.pallas import tpu_sc as plsc

B, CTX, D, V = 1024, 20, 64, 100000

NC, NS, L = 2, 16, 16
NW = NC * NS
BPW = B // NW
RPW = BPW * CTX
DP = 128

V_TILE = 2048
NV = (V + V_TILE - 1) // V_TILE
LAST_VALID = V - (NV - 1) * V_TILE


def _tc_prep_table(embT):
    def kern(e_ref, o_ref):
        o_ref[...] = jnp.concatenate(
            [
                jnp.transpose(e_ref[...], (1, 0)),
                jnp.zeros((V_TILE, DP - D), jnp.float32),
            ],
            axis=1,
        )

    return pl.pallas_call(
        kern,
        grid=(NV,),
        in_specs=[pl.BlockSpec((D, V_TILE), lambda i: (0, i))],
        out_specs=pl.BlockSpec((V_TILE, DP), lambda i: (i, 0)),
        out_shape=jax.ShapeDtypeStruct((V, DP), jnp.float32),
    )(embT)


def _sc_embed_sum(xT, embp):
    mesh = plsc.VectorSubcoreMesh(core_axis_name="c", subcore_axis_name="s")

    @functools.partial(
        pl.kernel,
        mesh=mesh,
        out_type=jax.ShapeDtypeStruct((B, D), jnp.float32),
        scratch_types=[
            pltpu.VMEM((CTX, B), jnp.int32),
            pltpu.VMEM((RPW, DP), jnp.float32),
            pltpu.VMEM((BPW, D), jnp.float32),
            pltpu.SemaphoreType.DMA,
        ],
    )
    def k(x_hbm, emb_hbm, out_hbm, idx_v, rows_v, acc_v, sem):
        wid = lax.axis_index("s") * NC + lax.axis_index("c")
        base = wid * BPW
        pltpu.sync_copy(x_hbm, idx_v)
        copies = [
            pltpu.async_copy(
                emb_hbm.at[idx_v.at[j, pl.ds(base, BPW)]],
                rows_v.at[pl.ds(j * BPW, BPW)],
                sem,
            )
            for j in range(CTX)
        ]
        for cp in copies:
            cp.wait()

        def body(bi, carry):
            for g in range(D // L):
                acc = rows_v[bi, pl.ds(g * L, L)]
                for j in range(1, CTX):
                    acc = acc + rows_v[j * BPW + bi, pl.ds(g * L, L)]
                acc_v[bi, pl.ds(g * L, L)] = acc
            return carry

        lax.fori_loop(0, BPW, body, 0)
        pltpu.sync_copy(acc_v, out_hbm.at[pl.ds(base, BPW)])

    return k(xT, embp)


def _tc_stats(s, xT, emb0, WT, b):
    def kern(s_ref, x_ref, e0_ref, w_ref, b_ref, m_ref, lse_ref, lrun):
        v = pl.program_id(0)

        @pl.when(v == 0)
        def _():
            n0row = jnp.sum(
                jnp.where(x_ref[...] == 0, 1.0, 0.0), axis=0, keepdims=True
            )
            n0 = jnp.transpose(n0row, (1, 0))
            m_ref[...] = (s_ref[...] - n0 * e0_ref[...]) * (1.0 / CTX)
            lrun[...] = jnp.zeros((1, B), jnp.float32)

        bcol = jnp.transpose(b_ref[...], (1, 0))
        logits = (
            lax.dot_general(
                w_ref[...],
                m_ref[...],
                (((0,), (1,)), ((), ())),
                preferred_element_type=jnp.float32,
            )
            + bcol
        )
        ex = jnp.exp(logits)

        @pl.when(v < NV - 1)
        def _():
            lrun[...] = lrun[...] + jnp.sum(ex, axis=0, keepdims=True)

        @pl.when(v == NV - 1)
        def _():
            row = lax.broadcasted_iota(jnp.int32, (V_TILE, 1), 0)
            exm = jnp.where(row < LAST_VALID, ex, 0.0)
            lse_ref[...] = jnp.log(
                lrun[...] + jnp.sum(exm, axis=0, keepdims=True)
            )

    return pl.pallas_call(
        kern,
        grid=(NV,),
        in_specs=[
            pl.BlockSpec((B, D), lambda v: (0, 0)),
            pl.BlockSpec((CTX, B), lambda v: (0, 0)),
            pl.BlockSpec((1, D), lambda v: (0, 0)),
            pl.BlockSpec((D, V_TILE), lambda v: (0, v)),
            pl.BlockSpec((1, V_TILE), lambda v: (0, v)),
        ],
        out_specs=[
            pl.BlockSpec((B, D), lambda v: (0, 0)),
            pl.BlockSpec((1, B), lambda v: (0, 0)),
        ],
        out_shape=[
            jax.ShapeDtypeStruct((B, D), jnp.float32),
            jax.ShapeDtypeStruct((1, B), jnp.float32),
        ],
        scratch_shapes=[pltpu.VMEM((1, B), jnp.float32)],
    )(s, xT, emb0, WT, b)


def _tc_out(m, WT, b, lse):
    def kern(m_ref, w_ref, b_ref, l_ref, o_ref):
        bcol = jnp.transpose(b_ref[...], (1, 0))
        logits = (
            lax.dot_general(
                w_ref[...],
                m_ref[...],
                (((0,), (1,)), ((), ())),
                preferred_element_type=jnp.float32,
            )
            + bcol
        )
        o_ref[...] = logits - l_ref[...]

    return pl.pallas_call(
        kern,
        grid=(NV,),
        in_specs=[
            pl.BlockSpec((B, D), lambda v: (0, 0)),
            pl.BlockSpec((D, V_TILE), lambda v: (0, v)),
            pl.BlockSpec((1, V_TILE), lambda v: (0, v)),
            pl.BlockSpec((1, B), lambda v: (0, 0)),
        ],
        out_specs=pl.BlockSpec((V_TILE, B), lambda v: (v, 0)),
        out_shape=jax.ShapeDtypeStruct((V, B), jnp.float32),
    )(m, WT, b, lse)


def kernel(x, emb, W, b):
    x32 = x.astype(jnp.int32)
    xT = x32.T
    embT = emb.T
    WT = W.T
    b2 = b[None, :]
    embp = _tc_prep_table(embT)
    s = _sc_embed_sum(xT, embp)
    m, lse = _tc_stats(s, xT, emb[0:1, :], WT, b2)
    outT = _tc_out(m, WT, b2, lse)
    return outT.T[:, None, :]


# speedup vs baseline: 2.3187x; 1.0650x over previous
import functools

import jax
import jax.numpy as jnp
from jax import lax
from jax.experimental import pallas as pl
from jax.experimental.pallas import tpu as pltpu
from jax.experimental.pallas import tpu_sc as plsc

B, CTX, D, V = 1024, 20, 64, 100000

NC, NS, L = 2, 16, 16
NW = NC * NS
BPW = B // NW
RPW = BPW * CTX
DP = 128

V_TILE = 2048
NV = (V + V_TILE - 1) // V_TILE
LAST_VALID = V - (NV - 1) * V_TILE


def _tc_prep_table(embT):
    def kern(e_ref, o_ref):
        o_ref[...] = jnp.concatenate(
            [
                jnp.transpose(e_ref[...], (1, 0)),
                jnp.zeros((V_TILE, DP - D), jnp.float32),
            ],
            axis=1,
        )

    return pl.pallas_call(
        kern,
        grid=(NV,),
        in_specs=[pl.BlockSpec((D, V_TILE), lambda i: (0, i))],
        out_specs=pl.BlockSpec((V_TILE, DP), lambda i: (i, 0)),
        out_shape=jax.ShapeDtypeStruct((V, DP), jnp.float32),
    )(embT)


def _sc_embed_sum(xT, embp):
    mesh = plsc.VectorSubcoreMesh(core_axis_name="c", subcore_axis_name="s")

    @functools.partial(
        pl.kernel,
        mesh=mesh,
        out_type=jax.ShapeDtypeStruct((B, D), jnp.float32),
        scratch_types=[
            pltpu.VMEM((CTX, B), jnp.int32),
            pltpu.VMEM((RPW, DP), jnp.float32),
            pltpu.VMEM((BPW, D), jnp.float32),
            pltpu.SemaphoreType.DMA,
        ],
    )
    def k(x_hbm, emb_hbm, out_hbm, idx_v, rows_v, acc_v, sem):
        wid = lax.axis_index("s") * NC + lax.axis_index("c")
        base = wid * BPW
        pltpu.sync_copy(x_hbm, idx_v)
        copies = [
            pltpu.async_copy(
                emb_hbm.at[idx_v.at[j, pl.ds(base, BPW)]],
                rows_v.at[pl.ds(j * BPW, BPW)],
                sem,
            )
            for j in range(CTX)
        ]
        for cp in copies:
            cp.wait()

        def body(bi, carry):
            for g in range(D // L):
                acc = rows_v[bi, pl.ds(g * L, L)]
                for j in range(1, CTX):
                    acc = acc + rows_v[j * BPW + bi, pl.ds(g * L, L)]
                acc_v[bi, pl.ds(g * L, L)] = acc
            return carry

        lax.fori_loop(0, BPW, body, 0)
        pltpu.sync_copy(acc_v, out_hbm.at[pl.ds(base, BPW)])

    return k(xT, embp)


def _tc_stats(s, xT, emb0, WT, b):
    def kern(s_ref, x_ref, e0_ref, w_ref, b_ref, m_ref, lse_ref, lrun):
        v = pl.program_id(0)

        @pl.when(v == 0)
        def _():
            n0row = jnp.sum(
                jnp.where(x_ref[...] == 0, 1.0, 0.0), axis=0, keepdims=True
            )
            n0 = jnp.transpose(n0row, (1, 0))
            m_ref[...] = (s_ref[...] - n0 * e0_ref[...]) * (1.0 / CTX)
            lrun[...] = jnp.zeros((1, B), jnp.float32)

        ex = jnp.exp(
            lax.dot_general(
                w_ref[...],
                m_ref[...],
                (((0,), (1,)), ((), ())),
                preferred_element_type=jnp.float32,
            )
        )
        col = lax.broadcasted_iota(jnp.int32, (1, V_TILE), 1)
        eb = jnp.where(col + v * V_TILE < V, jnp.exp(b_ref[...]), 0.0)
        lrun[...] = lrun[...] + lax.dot_general(
            eb,
            ex,
            (((1,), (0,)), ((), ())),
            preferred_element_type=jnp.float32,
        )

        @pl.when(v == NV - 1)
        def _():
            lse_ref[...] = jnp.log(lrun[...])

    return pl.pallas_call(
        kern,
        grid=(NV,),
        in_specs=[
            pl.BlockSpec((B, D), lambda v: (0, 0)),
            pl.BlockSpec((CTX, B), lambda v: (0, 0)),
            pl.BlockSpec((1, D), lambda v: (0, 0)),
            pl.BlockSpec((D, V_TILE), lambda v: (0, v)),
            pl.BlockSpec((1, V_TILE), lambda v: (0, v)),
        ],
        out_specs=[
            pl.BlockSpec((B, D), lambda v: (0, 0)),
            pl.BlockSpec((1, B), lambda v: (0, 0)),
        ],
        out_shape=[
            jax.ShapeDtypeStruct((B, D), jnp.float32),
            jax.ShapeDtypeStruct((1, B), jnp.float32),
        ],
        scratch_shapes=[pltpu.VMEM((1, B), jnp.float32)],
    )(s, xT, emb0, WT, b)


def _tc_out(m, WT, b, lse):
    def kern(m_ref, w_ref, b_ref, l_ref, o_ref):
        bcol = jnp.transpose(b_ref[...], (1, 0))
        logits = (
            lax.dot_general(
                w_ref[...],
                m_ref[...],
                (((0,), (1,)), ((), ())),
                preferred_element_type=jnp.float32,
            )
            + bcol
        )
        o_ref[...] = logits - l_ref[...]

    return pl.pallas_call(
        kern,
        grid=(NV,),
        in_specs=[
            pl.BlockSpec((B, D), lambda v: (0, 0)),
            pl.BlockSpec((D, V_TILE), lambda v: (0, v)),
            pl.BlockSpec((1, V_TILE), lambda v: (0, v)),
            pl.BlockSpec((1, B), lambda v: (0, 0)),
        ],
        out_specs=pl.BlockSpec((V_TILE, B), lambda v: (v, 0)),
        out_shape=jax.ShapeDtypeStruct((V, B), jnp.float32),
    )(m, WT, b, lse)


def kernel(x, emb, W, b):
    x32 = x.astype(jnp.int32)
    xT = x32.T
    embT = emb.T
    WT = W.T
    b2 = b[None, :]
    embp = _tc_prep_table(embT)
    s = _sc_embed_sum(xT, embp)
    m, lse = _tc_stats(s, xT, emb[0:1, :], WT, b2)
    outT = _tc_out(m, WT, b2, lse)
    return outT.T[:, None, :]


# speedup vs baseline: 2.4560x; 1.0592x over previous
import functools

import jax
import jax.numpy as jnp
from jax import lax
from jax.experimental import pallas as pl
from jax.experimental.pallas import tpu as pltpu
from jax.experimental.pallas import tpu_sc as plsc

B, CTX, D, V = 1024, 20, 64, 100000

NC, NS, L = 2, 16, 16
NW = NC * NS
BPW = B // NW
RPW = BPW * CTX
DP = 128

V_TILE = 4096
NV = (V + V_TILE - 1) // V_TILE
LAST_VALID = V - (NV - 1) * V_TILE


def _tc_prep_table(embT):
    def kern(e_ref, o_ref):
        o_ref[...] = jnp.concatenate(
            [
                jnp.transpose(e_ref[...], (1, 0)),
                jnp.zeros((V_TILE, DP - D), jnp.float32),
            ],
            axis=1,
        )

    return pl.pallas_call(
        kern,
        grid=(NV,),
        in_specs=[pl.BlockSpec((D, V_TILE), lambda i: (0, i))],
        out_specs=pl.BlockSpec((V_TILE, DP), lambda i: (i, 0)),
        out_shape=jax.ShapeDtypeStruct((V, DP), jnp.float32),
    )(embT)


def _sc_embed_sum(xT, embp):
    mesh = plsc.VectorSubcoreMesh(core_axis_name="c", subcore_axis_name="s")

    @functools.partial(
        pl.kernel,
        mesh=mesh,
        out_type=jax.ShapeDtypeStruct((B, D), jnp.float32),
        scratch_types=[
            pltpu.VMEM((CTX, B), jnp.int32),
            pltpu.VMEM((RPW, DP), jnp.float32),
            pltpu.VMEM((BPW, D), jnp.float32),
            pltpu.SemaphoreType.DMA,
        ],
    )
    def k(x_hbm, emb_hbm, out_hbm, idx_v, rows_v, acc_v, sem):
        wid = lax.axis_index("s") * NC + lax.axis_index("c")
        base = wid * BPW
        pltpu.sync_copy(x_hbm, idx_v)
        copies = [
            pltpu.async_copy(
                emb_hbm.at[idx_v.at[j, pl.ds(base, BPW)]],
                rows_v.at[pl.ds(j * BPW, BPW)],
                sem,
            )
            for j in range(CTX)
        ]
        for cp in copies:
            cp.wait()

        def body(bi, carry):
            for g in range(D // L):
                acc = rows_v[bi, pl.ds(g * L, L)]
                for j in range(1, CTX):
                    acc = acc + rows_v[j * BPW + bi, pl.ds(g * L, L)]
                acc_v[bi, pl.ds(g * L, L)] = acc
            return carry

        lax.fori_loop(0, BPW, body, 0)
        pltpu.sync_copy(acc_v, out_hbm.at[pl.ds(base, BPW)])

    return k(xT, embp)


def _tc_stats(s, xT, emb0, WT, b):
    def kern(s_ref, x_ref, e0_ref, w_ref, b_ref, m_ref, lse_ref, lrun):
        v = pl.program_id(0)

        @pl.when(v == 0)
        def _():
            n0row = jnp.sum(
                jnp.where(x_ref[...] == 0, 1.0, 0.0), axis=0, keepdims=True
            )
            n0 = jnp.transpose(n0row, (1, 0))
            m_ref[...] = (s_ref[...] - n0 * e0_ref[...]) * (1.0 / CTX)
            lrun[...] = jnp.zeros((1, B), jnp.float32)

        ex = jnp.exp(
            lax.dot_general(
                w_ref[...],
                m_ref[...],
                (((0,), (1,)), ((), ())),
                preferred_element_type=jnp.float32,
            )
        )
        col = lax.broadcasted_iota(jnp.int32, (1, V_TILE), 1)
        eb = jnp.where(col + v * V_TILE < V, jnp.exp(b_ref[...]), 0.0)
        lrun[...] = lrun[...] + lax.dot_general(
            eb,
            ex,
            (((1,), (0,)), ((), ())),
            preferred_element_type=jnp.float32,
        )

        @pl.when(v == NV - 1)
        def _():
            lse_ref[...] = jnp.log(lrun[...])

    return pl.pallas_call(
        kern,
        grid=(NV,),
        in_specs=[
            pl.BlockSpec((B, D), lambda v: (0, 0)),
            pl.BlockSpec((CTX, B), lambda v: (0, 0)),
            pl.BlockSpec((1, D), lambda v: (0, 0)),
            pl.BlockSpec((D, V_TILE), lambda v: (0, v)),
            pl.BlockSpec((1, V_TILE), lambda v: (0, v)),
        ],
        out_specs=[
            pl.BlockSpec((B, D), lambda v: (0, 0)),
            pl.BlockSpec((1, B), lambda v: (0, 0)),
        ],
        out_shape=[
            jax.ShapeDtypeStruct((B, D), jnp.float32),
            jax.ShapeDtypeStruct((1, B), jnp.float32),
        ],
        scratch_shapes=[pltpu.VMEM((1, B), jnp.float32)],
    )(s, xT, emb0, WT, b)


def _tc_out(m, WT, b, lse):
    def kern(m_ref, w_ref, b_ref, l_ref, o_ref):
        bcol = jnp.transpose(b_ref[...], (1, 0))
        logits = (
            lax.dot_general(
                w_ref[...],
                m_ref[...],
                (((0,), (1,)), ((), ())),
                preferred_element_type=jnp.float32,
            )
            + bcol
        )
        o_ref[...] = logits - l_ref[...]

    return pl.pallas_call(
        kern,
        grid=(NV,),
        in_specs=[
            pl.BlockSpec((B, D), lambda v: (0, 0)),
            pl.BlockSpec((D, V_TILE), lambda v: (0, v)),
            pl.BlockSpec((1, V_TILE), lambda v: (0, v)),
            pl.BlockSpec((1, B), lambda v: (0, 0)),
        ],
        out_specs=pl.BlockSpec((V_TILE, B), lambda v: (v, 0)),
        out_shape=jax.ShapeDtypeStruct((V, B), jnp.float32),
    )(m, WT, b, lse)


def kernel(x, emb, W, b):
    x32 = x.astype(jnp.int32)
    xT = x32.T
    embT = emb.T
    WT = W.T
    b2 = b[None, :]
    embp = _tc_prep_table(embT)
    s = _sc_embed_sum(xT, embp)
    m, lse = _tc_stats(s, xT, emb[0:1, :], WT, b2)
    outT = _tc_out(m, WT, b2, lse)
    return outT.T[:, None, :]
